# baseline (speedup 1.0000x reference)
import functools

import jax
import jax.numpy as jnp
from jax.experimental import pallas as pl
from jax.experimental.pallas import tpu as pltpu

TOKENS = 16384
INPUT_DIM = 2048
NUM_EXPERTS = 16
TOP_K = 2
BLOCK = 1024


def _gate_kernel(x_ref, wt_ref, b_ref, u_ref, idx_ref, gate_ref, var_ref):
    logits = jnp.dot(x_ref[:], wt_ref[:], preferred_element_type=jnp.float32)
    logits = logits + b_ref[:]
    m = jnp.max(logits, axis=1, keepdims=True)
    e = jnp.exp(logits - m)
    probs = e / jnp.sum(e, axis=1, keepdims=True)
    col = jax.lax.broadcasted_iota(jnp.int32, probs.shape, 1)
    m1 = jnp.max(probs, axis=1, keepdims=True)
    i1 = jnp.min(jnp.where(probs == m1, col, NUM_EXPERTS), axis=1, keepdims=True)
    probs2 = jnp.where(col == i1, -jnp.inf, probs)
    m2 = jnp.max(probs2, axis=1, keepdims=True)
    i2 = jnp.min(jnp.where(probs2 == m2, col, NUM_EXPERTS), axis=1, keepdims=True)
    s = m1 + m2 + 1e-8
    idx_ref[:] = jnp.concatenate([i1, i2], axis=1)
    gate_ref[:] = jnp.concatenate([m1 / s, m2 / s], axis=1)
    u = u_ref[:]
    mu = jnp.sum(u) / NUM_EXPERTS
    var_ref[:] = (jnp.sum((u - mu) ** 2) / (NUM_EXPERTS - 1)).reshape(1, 1)


@jax.jit
def kernel(x, W, b, expert_usage):
    wt = W.T
    b2 = b.reshape(1, NUM_EXPERTS)
    u2 = expert_usage.reshape(1, NUM_EXPERTS)
    grid = TOKENS // BLOCK
    idx, gates, var = pl.pallas_call(
        _gate_kernel,
        grid=(grid,),
        in_specs=[
            pl.BlockSpec((BLOCK, INPUT_DIM), lambda i: (i, 0)),
            pl.BlockSpec((INPUT_DIM, NUM_EXPERTS), lambda i: (0, 0)),
            pl.BlockSpec((1, NUM_EXPERTS), lambda i: (0, 0)),
            pl.BlockSpec((1, NUM_EXPERTS), lambda i: (0, 0)),
        ],
        out_specs=[
            pl.BlockSpec((BLOCK, TOP_K), lambda i: (i, 0)),
            pl.BlockSpec((BLOCK, TOP_K), lambda i: (i, 0)),
            pl.BlockSpec((1, 1), lambda i: (0, 0)),
        ],
        out_shape=[
            jax.ShapeDtypeStruct((TOKENS, TOP_K), jnp.int32),
            jax.ShapeDtypeStruct((TOKENS, TOP_K), jnp.float32),
            jax.ShapeDtypeStruct((1, 1), jnp.float32),
        ],
        compiler_params=pltpu.CompilerParams(
            dimension_semantics=("arbitrary",),
        ),
    )(x, wt, b2, u2)
    return idx, gates, var[0, 0]


# speedup vs baseline: 1.0207x; 1.0207x over previous
import functools

import jax
import jax.numpy as jnp
from jax.experimental import pallas as pl
from jax.experimental.pallas import tpu as pltpu

TOKENS = 16384
INPUT_DIM = 2048
NUM_EXPERTS = 16
TOP_K = 2
BLOCK = 1024


def _gate_kernel(x_ref, wt_ref, b_ref, u_ref, idx_ref, gate_ref, var_ref):
    logits = jnp.dot(x_ref[:], wt_ref[:], preferred_element_type=jnp.float32)
    logits = logits + b_ref[:]
    m = jnp.max(logits, axis=1, keepdims=True)
    e = jnp.exp(logits - m)
    z = jnp.sum(e, axis=1, keepdims=True)
    col = jax.lax.broadcasted_iota(jnp.int32, e.shape, 1)
    m1 = jnp.max(e, axis=1, keepdims=True)
    i1 = jnp.min(jnp.where(e == m1, col, NUM_EXPERTS), axis=1, keepdims=True)
    e2 = jnp.where(col == i1, -1.0, e)
    m2 = jnp.max(e2, axis=1, keepdims=True)
    i2 = jnp.min(jnp.where(e2 == m2, col, NUM_EXPERTS), axis=1, keepdims=True)
    s = m1 + m2 + 1e-8 * z
    idx_ref[:] = jnp.concatenate([i1, i2], axis=1)
    gate_ref[:] = jnp.concatenate([m1 / s, m2 / s], axis=1)
    u = u_ref[:]
    mu = jnp.sum(u) / NUM_EXPERTS
    var_ref[:] = (jnp.sum((u - mu) ** 2) / (NUM_EXPERTS - 1)).reshape(1, 1)


@jax.jit
def kernel(x, W, b, expert_usage):
    wt = W.T
    b2 = b.reshape(1, NUM_EXPERTS)
    u2 = expert_usage.reshape(1, NUM_EXPERTS)
    grid = TOKENS // BLOCK
    idx, gates, var = pl.pallas_call(
        _gate_kernel,
        grid=(grid,),
        in_specs=[
            pl.BlockSpec((BLOCK, INPUT_DIM), lambda i: (i, 0)),
            pl.BlockSpec((INPUT_DIM, NUM_EXPERTS), lambda i: (0, 0)),
            pl.BlockSpec((1, NUM_EXPERTS), lambda i: (0, 0)),
            pl.BlockSpec((1, NUM_EXPERTS), lambda i: (0, 0)),
        ],
        out_specs=[
            pl.BlockSpec((BLOCK, TOP_K), lambda i: (i, 0)),
            pl.BlockSpec((BLOCK, TOP_K), lambda i: (i, 0)),
            pl.BlockSpec((1, 1), lambda i: (0, 0)),
        ],
        out_shape=[
            jax.ShapeDtypeStruct((TOKENS, TOP_K), jnp.int32),
            jax.ShapeDtypeStruct((TOKENS, TOP_K), jnp.float32),
            jax.ShapeDtypeStruct((1, 1), jnp.float32),
        ],
        compiler_params=pltpu.CompilerParams(
            dimension_semantics=("parallel",),
        ),
    )(x, wt, b2, u2)
    return idx, gates, var[0, 0]


# speedup vs baseline: 1.0603x; 1.0387x over previous
import functools

import jax
import jax.numpy as jnp
from jax.experimental import pallas as pl
from jax.experimental.pallas import tpu as pltpu

TOKENS = 16384
INPUT_DIM = 2048
NUM_EXPERTS = 16
TOP_K = 2
BLOCK = 2048


def _gate_kernel(x_ref, wt_ref, b_ref, u_ref, idx_ref, gate_ref, var_ref):
    logits = jnp.dot(x_ref[:], wt_ref[:], preferred_element_type=jnp.float32)
    logits = logits + b_ref[:]
    m = jnp.max(logits, axis=1, keepdims=True)
    e = jnp.exp(logits - m)
    z = jnp.sum(e, axis=1, keepdims=True)
    col = jax.lax.broadcasted_iota(jnp.int32, e.shape, 1)
    m1 = jnp.max(e, axis=1, keepdims=True)
    i1 = jnp.min(jnp.where(e == m1, col, NUM_EXPERTS), axis=1, keepdims=True)
    e2 = jnp.where(col == i1, -1.0, e)
    m2 = jnp.max(e2, axis=1, keepdims=True)
    i2 = jnp.min(jnp.where(e2 == m2, col, NUM_EXPERTS), axis=1, keepdims=True)
    s = m1 + m2 + 1e-8 * z
    idx_ref[:] = jnp.concatenate([i1, i2], axis=1)
    gate_ref[:] = jnp.concatenate([m1 / s, m2 / s], axis=1)
    u = u_ref[:]
    mu = jnp.sum(u) / NUM_EXPERTS
    var_ref[:] = (jnp.sum((u - mu) ** 2) / (NUM_EXPERTS - 1)).reshape(1, 1)


@jax.jit
def kernel(x, W, b, expert_usage):
    wt = W.T
    b2 = b.reshape(1, NUM_EXPERTS)
    u2 = expert_usage.reshape(1, NUM_EXPERTS)
    grid = TOKENS // BLOCK
    idx, gates, var = pl.pallas_call(
        _gate_kernel,
        grid=(grid,),
        in_specs=[
            pl.BlockSpec((BLOCK, INPUT_DIM), lambda i: (i, 0)),
            pl.BlockSpec((INPUT_DIM, NUM_EXPERTS), lambda i: (0, 0)),
            pl.BlockSpec((1, NUM_EXPERTS), lambda i: (0, 0)),
            pl.BlockSpec((1, NUM_EXPERTS), lambda i: (0, 0)),
        ],
        out_specs=[
            pl.BlockSpec((BLOCK, TOP_K), lambda i: (i, 0)),
            pl.BlockSpec((BLOCK, TOP_K), lambda i: (i, 0)),
            pl.BlockSpec((1, 1), lambda i: (0, 0)),
        ],
        out_shape=[
            jax.ShapeDtypeStruct((TOKENS, TOP_K), jnp.int32),
            jax.ShapeDtypeStruct((TOKENS, TOP_K), jnp.float32),
            jax.ShapeDtypeStruct((1, 1), jnp.float32),
        ],
        compiler_params=pltpu.CompilerParams(
            dimension_semantics=("parallel",),
        ),
    )(x, wt, b2, u2)
    return idx, gates, var[0, 0]


# speedup vs baseline: 1.0841x; 1.0225x over previous
import jax
import jax.numpy as jnp
from jax.experimental import pallas as pl
from jax.experimental.pallas import tpu as pltpu

TOKENS = 16384
INPUT_DIM = 2048
NUM_EXPERTS = 16
TOP_K = 2
BLOCK = 2048
SUB = 256


def _gate_kernel(x_ref, wt_ref, b_ref, u_ref, idx_ref, gate_ref, var_ref):
    for j in range(BLOCK // SUB):
        sl = pl.ds(j * SUB, SUB)
        logits = jnp.dot(x_ref[sl, :], wt_ref[:], preferred_element_type=jnp.float32)
        logits = logits + b_ref[:]
        col = jax.lax.broadcasted_iota(jnp.int32, logits.shape, 1)
        m1 = jnp.max(logits, axis=1, keepdims=True)
        i1 = jnp.min(jnp.where(logits == m1, col, NUM_EXPERTS), axis=1, keepdims=True)
        masked = jnp.where(col == i1, -jnp.inf, logits)
        m2 = jnp.max(masked, axis=1, keepdims=True)
        i2 = jnp.min(jnp.where(masked == m2, col, NUM_EXPERTS), axis=1, keepdims=True)
        z = jnp.sum(jnp.exp(logits - m1), axis=1, keepdims=True)
        e2 = jnp.exp(m2 - m1)
        g1 = 1.0 / (1.0 + e2 + 1e-8 * z)
        idx_ref[sl, :] = jnp.concatenate([i1, i2], axis=1)
        gate_ref[sl, :] = jnp.concatenate([g1, e2 * g1], axis=1)
    u = u_ref[:]
    mu = jnp.sum(u) / NUM_EXPERTS
    var_ref[:] = (jnp.sum((u - mu) ** 2) / (NUM_EXPERTS - 1)).reshape(1, 1)


@jax.jit
def kernel(x, W, b, expert_usage):
    wt = W.T
    b2 = b.reshape(1, NUM_EXPERTS)
    u2 = expert_usage.reshape(1, NUM_EXPERTS)
    grid = TOKENS // BLOCK
    idx, gates, var = pl.pallas_call(
        _gate_kernel,
        grid=(grid,),
        in_specs=[
            pl.BlockSpec((BLOCK, INPUT_DIM), lambda i: (i, 0)),
            pl.BlockSpec((INPUT_DIM, NUM_EXPERTS), lambda i: (0, 0)),
            pl.BlockSpec((1, NUM_EXPERTS), lambda i: (0, 0)),
            pl.BlockSpec((1, NUM_EXPERTS), lambda i: (0, 0)),
        ],
        out_specs=[
            pl.BlockSpec((BLOCK, TOP_K), lambda i: (i, 0)),
            pl.BlockSpec((BLOCK, TOP_K), lambda i: (i, 0)),
            pl.BlockSpec((1, 1), lambda i: (0, 0)),
        ],
        out_shape=[
            jax.ShapeDtypeStruct((TOKENS, TOP_K), jnp.int32),
            jax.ShapeDtypeStruct((TOKENS, TOP_K), jnp.float32),
            jax.ShapeDtypeStruct((1, 1), jnp.float32),
        ],
        compiler_params=pltpu.CompilerParams(
            dimension_semantics=("parallel",),
        ),
    )(x, wt, b2, u2)
    return idx, gates, var[0, 0]


# speedup vs baseline: 1.0841x; 1.0000x over previous
import jax
import jax.numpy as jnp
from jax.experimental import pallas as pl
from jax.experimental.pallas import tpu as pltpu

TOKENS = 16384
INPUT_DIM = 2048
NUM_EXPERTS = 16
TOP_K = 2
BLOCK = 2048
SUB = 256


def _gate_kernel(x_ref, wt_ref, b_ref, u_ref, idx_ref, gate_ref, var_ref):
    base = pl.program_id(0) * BLOCK
    for j in range(BLOCK // SUB):
        sl = pl.ds(j * SUB, SUB)
        osl = pl.ds(base + j * SUB, SUB)
        logits = jnp.dot(x_ref[sl, :], wt_ref[:], preferred_element_type=jnp.float32)
        logits = logits + b_ref[:]
        col = jax.lax.broadcasted_iota(jnp.int32, logits.shape, 1)
        m1 = jnp.max(logits, axis=1, keepdims=True)
        i1 = jnp.min(jnp.where(logits == m1, col, NUM_EXPERTS), axis=1, keepdims=True)
        masked = jnp.where(col == i1, -jnp.inf, logits)
        m2 = jnp.max(masked, axis=1, keepdims=True)
        i2 = jnp.min(jnp.where(masked == m2, col, NUM_EXPERTS), axis=1, keepdims=True)
        z = jnp.sum(jnp.exp(logits - m1), axis=1, keepdims=True)
        e2 = jnp.exp(m2 - m1)
        g1 = 1.0 / (1.0 + e2 + 1e-8 * z)
        idx_ref[osl, :] = jnp.concatenate([i1, i2], axis=1)
        gate_ref[osl, :] = jnp.concatenate([g1, e2 * g1], axis=1)
    u = u_ref[:]
    mu = jnp.sum(u) / NUM_EXPERTS
    var_ref[:] = (jnp.sum((u - mu) ** 2) / (NUM_EXPERTS - 1)).reshape(1, 1)


@jax.jit
def kernel(x, W, b, expert_usage):
    wt = W.T
    b2 = b.reshape(1, NUM_EXPERTS)
    u2 = expert_usage.reshape(1, NUM_EXPERTS)
    grid = TOKENS // BLOCK
    idx, gates, var = pl.pallas_call(
        _gate_kernel,
        grid=(grid,),
        in_specs=[
            pl.BlockSpec((BLOCK, INPUT_DIM), lambda i: (i, 0)),
            pl.BlockSpec((INPUT_DIM, NUM_EXPERTS), lambda i: (0, 0)),
            pl.BlockSpec((1, NUM_EXPERTS), lambda i: (0, 0)),
            pl.BlockSpec((1, NUM_EXPERTS), lambda i: (0, 0)),
        ],
        out_specs=[
            pl.BlockSpec((TOKENS, TOP_K), lambda i: (0, 0)),
            pl.BlockSpec((TOKENS, TOP_K), lambda i: (0, 0)),
            pl.BlockSpec((1, 1), lambda i: (0, 0)),
        ],
        out_shape=[
            jax.ShapeDtypeStruct((TOKENS, TOP_K), jnp.int32),
            jax.ShapeDtypeStruct((TOKENS, TOP_K), jnp.float32),
            jax.ShapeDtypeStruct((1, 1), jnp.float32),
        ],
        compiler_params=pltpu.CompilerParams(
            dimension_semantics=("parallel",),
        ),
    )(x, wt, b2, u2)
    return idx, gates, var[0, 0]
